# baseline (speedup 1.0000x reference)
import functools

import jax
import jax.numpy as jnp
from jax import lax
from jax.experimental import pallas as pl
from jax.experimental.pallas import tpu as pltpu
from jax.experimental.pallas import tpu_sc as plsc

B = 64
N = 8192
K = 4096
L = 16
NC = 2
NS = 16
NW = NC * NS
ROWS_PER_W = B // NW
TOPBIT = jnp.int32(-(2**31))


def _sc_body(x_hbm, out_hbm, xrow0, xrow1, outrow0, outrow1, cand, hist, sems):
    wid = lax.axis_index("s") * NC + lax.axis_index("c")
    iota = lax.broadcasted_iota(jnp.int32, (L,), 0)
    zeros_i = jnp.zeros((L,), jnp.int32)
    ones_i = jnp.ones((L,), jnp.int32)
    sh24 = jnp.full((L,), 24, jnp.int32)
    sh31 = jnp.full((L,), 31, jnp.int32)

    def process_row(xrow, outrow):
        with jax.named_scope("zero"):
            @plsc.parallel_loop(0, 8 * 256, step=L, unroll=8)
            def _(j):
                hist[pl.ds(j, L)] = zeros_i

        def hist_pass(lo, hi):
            @plsc.parallel_loop(lo, hi, step=L, unroll=8)
            def _(i):
                xv = xrow[pl.ds(i, L)]
                bits = plsc.bitcast(xv, jnp.int32)
                key = bits ^ (lax.shift_right_arithmetic(bits, sh31) | TOPBIT)
                bucket = lax.shift_right_logical(key, sh24)
                slot_base = lax.shift_left(i, 4) & (7 << 8)
                plsc.addupdate_scatter(hist, [slot_base + bucket], ones_i)
        with jax.named_scope("hist"):
            hist_pass(0, N)

        def sbody(i, carry):
            cum, b_star, above = carry
            v = 15 - i
            acc = zeros_i
            for l in range(8):
                acc = acc + hist[pl.ds(l * 256 + v * L, L)]
            s = lax.rev(jnp.cumsum(lax.rev(acc, (0,))), (0,))
            cnt_ge = cum + s
            mask = cnt_ge >= K
            b_loc = jnp.max(jnp.where(mask, v * L + iota, -1))
            a_loc = jnp.min(jnp.where(mask, cnt_ge - acc, jnp.int32(2**31 - 1)))
            better = b_loc > b_star
            b_star = jnp.where(better, b_loc, b_star)
            above = jnp.where(better, a_loc, above)
            cum = cum + jnp.sum(acc)
            return cum, b_star, above
        with jax.named_scope("scan"):
            _, b1, above = lax.fori_loop(
                0, 16, sbody, (jnp.int32(0), jnp.int32(-1), jnp.int32(0)))
        kk = jnp.int32(K) - above

        with jax.named_scope("compact"):
            @plsc.parallel_loop(0, N, step=L, unroll=4,
                                carry=jnp.full((L,), -1, jnp.int32))
            def offm1(i, offm1):
                xv = xrow[pl.ds(i, L)]
                bits = plsc.bitcast(xv, jnp.int32)
                kv = bits ^ (lax.shift_right_arithmetic(bits, sh31) | TOPBIT)
                bucket = lax.shift_right_logical(kv, sh24)
                mk = bucket == b1
                pos = jnp.maximum(offm1 + jnp.cumsum(mk.astype(jnp.int32)), 0)
                plsc.store_scatter(cand, [pos], kv, mask=mk)
                return offm1 + plsc.all_reduce_population_count(mk)
            m_splat = offm1 + 1
            t0 = lax.shift_left(zeros_i + b1, sh24)
            for k in range(8):
                plsc.store_scatter(cand, [m_splat + (k * L + iota)], t0)
            m = jnp.max(m_splat)

        m_r = ((m + L - 1) // L) * L
        kk_splat = zeros_i + kk
        bm0 = jnp.full((L,), 1 << 23, jnp.int32)
        with jax.named_scope("bits"):
            def bits_fast(_):
                cv = tuple(cand[pl.ds(k * L, L)] for k in range(8))
                def fbody(i, carry):
                    t, bmask = carry
                    tp = t | bmask
                    cnt = zeros_i
                    for k in range(8):
                        cnt = cnt + plsc.all_reduce_population_count(
                            cv[k] >= tp)
                    t = jnp.where(cnt >= kk_splat, tp, t)
                    return t, lax.shift_right_logical(bmask, ones_i)
                t, _ = lax.fori_loop(0, 24, fbody, (t0, bm0))
                return t
            def bits_slow(_):
                def bitbody(i, carry):
                    t, bmask = carry
                    tp = t | bmask
                    @plsc.parallel_loop(0, m_r, step=L, unroll=2,
                                        carry=zeros_i)
                    def cnt(j, c):
                        kv = cand[pl.ds(j, L)]
                        return c + plsc.all_reduce_population_count(kv >= tp)
                    t = jnp.where(cnt >= kk_splat, tp, t)
                    return t, lax.shift_right_logical(bmask, ones_i)
                t, _ = lax.fori_loop(0, 24, bitbody, (t0, bm0))
                return t
            tsplat = lax.cond(m <= 7 * L, bits_fast, bits_slow, 0)

        fbits = jnp.where(tsplat < 0, tsplat ^ TOPBIT, ~tsplat)
        tvec = plsc.bitcast(fbits, jnp.float32)

        with jax.named_scope("mask"):
            @plsc.parallel_loop(0, N, step=L, unroll=8,
                                carry=jnp.zeros((L,), jnp.float32))
            def acc(i, a):
                xv = xrow[pl.ds(i, L)]
                rv = jnp.where(xv >= tvec, xv, jnp.float32(0))
                outrow[pl.ds(i, L)] = rv
                return a + rv

        s_splat = jnp.zeros((L,), jnp.float32) + jnp.sum(acc)
        scale = jnp.full((L,), N, jnp.float32) / s_splat

        with jax.named_scope("scale"):
            @plsc.parallel_loop(0, N, step=L, unroll=8)
            def _(i):
                outrow[pl.ds(i, L)] = outrow[pl.ds(i, L)] * scale

    row0 = wid * ROWS_PER_W
    with jax.named_scope("dma_in"):
        in0 = pltpu.async_copy(x_hbm.at[row0], xrow0, sems.at[0])
        in1 = pltpu.async_copy(x_hbm.at[row0 + 1], xrow1, sems.at[1])
        in0.wait()
    process_row(xrow0, outrow0)
    with jax.named_scope("dma_out"):
        out0 = pltpu.async_copy(outrow0, out_hbm.at[row0], sems.at[2])
        in1.wait()
    process_row(xrow1, outrow1)
    with jax.named_scope("dma_out"):
        out1 = pltpu.async_copy(outrow1, out_hbm.at[row0 + 1], sems.at[3])
        out0.wait()
        out1.wait()


def kernel(x):
    mesh = plsc.VectorSubcoreMesh(core_axis_name="c", subcore_axis_name="s")
    f = functools.partial(
        pl.kernel,
        mesh=mesh,
        compiler_params=pltpu.CompilerParams(
            needs_layout_passes=False, skip_device_barrier=True),
        out_type=jax.ShapeDtypeStruct((B, N), jnp.float32),
        scratch_types=[
            pltpu.VMEM((N,), jnp.float32),
            pltpu.VMEM((N,), jnp.float32),
            pltpu.VMEM((N,), jnp.float32),
            pltpu.VMEM((N,), jnp.float32),
            pltpu.VMEM((N + 8 * L,), jnp.int32),
            pltpu.VMEM((8 * 256,), jnp.int32),
            pltpu.SemaphoreType.DMA((4,)),
        ],
    )(_sc_body)
    return f(x)


# speedup vs baseline: 1.0321x; 1.0321x over previous
import functools

import jax
import jax.numpy as jnp
from jax import lax
from jax.experimental import pallas as pl
from jax.experimental.pallas import tpu as pltpu
from jax.experimental.pallas import tpu_sc as plsc

B = 64
N = 8192
K = 4096
L = 16
NC = 2
NS = 16
NW = NC * NS
ROWS_PER_W = B // NW
NSLOT = 8
TOPBIT = jnp.int32(-(2**31))


def _sc_body(x_hbm, out_hbm, xrow0, xrow1, outrow0, outrow1, keys, cand,
             hist, sems):
    wid = lax.axis_index("s") * NC + lax.axis_index("c")
    iota = lax.broadcasted_iota(jnp.int32, (L,), 0)
    zeros_i = jnp.zeros((L,), jnp.int32)
    ones_i = jnp.ones((L,), jnp.int32)
    sh24 = jnp.full((L,), 24, jnp.int32)
    sh31 = jnp.full((L,), 31, jnp.int32)

    def process_row(xrow, outrow):
        @plsc.parallel_loop(0, N, step=L, unroll=NSLOT)
        def _(i):
            xv = xrow[pl.ds(i, L)]
            bits = plsc.bitcast(xv, jnp.int32)
            key = bits ^ (lax.shift_right_arithmetic(bits, sh31) | TOPBIT)
            keys[pl.ds(i, L)] = key
            bucket = lax.shift_right_logical(key, sh24)
            slot_base = lax.shift_left(i, 4) & ((NSLOT - 1) << 8)
            plsc.addupdate_scatter(hist, [slot_base + bucket], ones_i)

        def sbody(i, carry):
            cum, b_star, above = carry
            v = 15 - i
            acc = zeros_i
            for s in range(NSLOT):
                acc = acc + hist[pl.ds(s * 256 + v * L, L)]
            for s in range(NSLOT):
                hist[pl.ds(s * 256 + v * L, L)] = zeros_i
            sfx = lax.rev(jnp.cumsum(lax.rev(acc, (0,))), (0,))
            cnt_ge = cum + sfx
            mask = cnt_ge >= K
            b_loc = jnp.max(jnp.where(mask, v * L + iota, -1))
            a_loc = jnp.min(jnp.where(mask, cnt_ge - acc, jnp.int32(2**31 - 1)))
            better = b_loc > b_star
            b_star = jnp.where(better, b_loc, b_star)
            above = jnp.where(better, a_loc, above)
            cum = cum + jnp.sum(acc)
            return cum, b_star, above
        _, b1, above = lax.fori_loop(
            0, 16, sbody, (jnp.int32(0), jnp.int32(-1), jnp.int32(0)))
        kk = jnp.int32(K) - above

        @plsc.parallel_loop(0, N, step=L, unroll=4,
                            carry=jnp.full((L,), -1, jnp.int32))
        def offm1(i, offm1):
            kv = keys[pl.ds(i, L)]
            bucket = lax.shift_right_logical(kv, sh24)
            mk = bucket == b1
            pos = jnp.maximum(offm1 + jnp.cumsum(mk.astype(jnp.int32)), 0)
            plsc.store_scatter(cand, [pos], kv, mask=mk)
            return offm1 + plsc.all_reduce_population_count(mk)
        m_splat = offm1 + 1
        t0 = lax.shift_left(zeros_i + b1, sh24)
        plsc.store_scatter(cand, [m_splat + iota], t0)
        m = jnp.max(m_splat)

        m_r = ((m + L - 1) // L) * L
        kk_splat = zeros_i + kk

        def bitbody(i, carry):
            t, bmask = carry
            tp = t | bmask
            @plsc.parallel_loop(0, m_r, step=L, unroll=2, carry=zeros_i)
            def cnt(j, c):
                kv = cand[pl.ds(j, L)]
                return c + plsc.all_reduce_population_count(kv >= tp)
            t = jnp.where(cnt >= kk_splat, tp, t)
            return t, lax.shift_right_logical(bmask, ones_i)
        tsplat, _ = lax.fori_loop(
            0, 24, bitbody, (t0, jnp.full((L,), 1 << 23, jnp.int32)))

        fbits = jnp.where(tsplat < 0, tsplat ^ TOPBIT, ~tsplat)
        tvec = plsc.bitcast(fbits, jnp.float32)

        @plsc.parallel_loop(0, N, step=L, unroll=NSLOT,
                            carry=jnp.zeros((L,), jnp.float32))
        def acc(i, a):
            xv = xrow[pl.ds(i, L)]
            rv = jnp.where(xv >= tvec, xv, jnp.float32(0))
            outrow[pl.ds(i, L)] = rv
            return a + rv

        s_splat = jnp.zeros((L,), jnp.float32) + jnp.sum(acc)
        scale = jnp.full((L,), N, jnp.float32) / s_splat

        @plsc.parallel_loop(0, N, step=L, unroll=NSLOT)
        def _(i):
            outrow[pl.ds(i, L)] = outrow[pl.ds(i, L)] * scale

    row0 = wid * ROWS_PER_W
    in0 = pltpu.async_copy(x_hbm.at[row0], xrow0, sems.at[0])
    in1 = pltpu.async_copy(x_hbm.at[row0 + 1], xrow1, sems.at[1])

    @plsc.parallel_loop(0, NSLOT * 256, step=L, unroll=8)
    def _(j):
        hist[pl.ds(j, L)] = zeros_i

    in0.wait()
    process_row(xrow0, outrow0)
    out0 = pltpu.async_copy(outrow0, out_hbm.at[row0], sems.at[2])
    in1.wait()
    process_row(xrow1, outrow1)
    out1 = pltpu.async_copy(outrow1, out_hbm.at[row0 + 1], sems.at[3])
    out0.wait()
    out1.wait()


def kernel(x):
    mesh = plsc.VectorSubcoreMesh(core_axis_name="c", subcore_axis_name="s")
    f = functools.partial(
        pl.kernel,
        mesh=mesh,
        compiler_params=pltpu.CompilerParams(
            needs_layout_passes=False, skip_device_barrier=True),
        out_type=jax.ShapeDtypeStruct((B, N), jnp.float32),
        scratch_types=[
            pltpu.VMEM((N,), jnp.float32),
            pltpu.VMEM((N,), jnp.float32),
            pltpu.VMEM((N,), jnp.float32),
            pltpu.VMEM((N,), jnp.float32),
            pltpu.VMEM((N,), jnp.int32),
            pltpu.VMEM((N + L,), jnp.int32),
            pltpu.VMEM((NSLOT * 256,), jnp.int32),
            pltpu.SemaphoreType.DMA((4,)),
        ],
    )(_sc_body)
    return f(x)
